# Initial kernel scaffold; baseline (speedup 1.0000x reference)
import functools

import jax
import jax.numpy as jnp
from jax import lax
from jax.experimental import pallas as pl
from jax.experimental.pallas import tpu as pltpu
from jax.experimental.pallas import tpu_sc as plsc

N = 10000
E = 320000
D = 128
G = 64

NC = 2
NS = 16
NW = NC * NS

NPAD = 10240
EPAD = 327680
EPT = EPAD // NW
ECH = 128
NCH = EPT // ECH
ROWS_PER_TILE = NPAD // NS
WB = ROWS_PER_TILE // ECH

_mesh = plsc.VectorSubcoreMesh(core_axis_name="c", subcore_axis_name="s")


HCH = 512


@functools.partial(
    pl.kernel,
    out_type=jax.ShapeDtypeStruct((NW, NPAD), jnp.float32),
    mesh=_mesh,
    scratch_types=[
        pltpu.VMEM((HCH,), jnp.int32),
        pltpu.VMEM((NPAD,), jnp.float32),
    ],
)
def _hist(dst_hbm, cnt_hbm, didx_v, cnt_v):
    c = lax.axis_index("c")
    s = lax.axis_index("s")
    wid = c * NS + s
    base = wid * EPT

    zeros16 = jnp.zeros((16,), jnp.float32)
    ones16 = jnp.ones((16,), jnp.float32)

    def zero_body(i, carry):
        cnt_v[pl.ds(i * 16, 16)] = zeros16
        return carry

    lax.fori_loop(0, NPAD // 16, zero_body, 0)

    def chunk_body(k, carry):
        pltpu.sync_copy(dst_hbm.at[pl.ds(base + k * HCH, HCH)], didx_v)

        def inner(j, carry2):
            iv = didx_v[pl.ds(j * 16, 16)]
            plsc.addupdate_scatter(cnt_v, [iv], ones16)
            return carry2

        lax.fori_loop(0, HCH // 16, inner, 0)
        return carry

    lax.fori_loop(0, EPT // HCH, chunk_body, 0)
    pltpu.sync_copy(cnt_v, cnt_hbm.at[wid])


@functools.partial(
    pl.kernel,
    out_type=jax.ShapeDtypeStruct((NC, NPAD, D), jnp.float32),
    mesh=_mesh,
    scratch_types=[
        pltpu.VMEM((2, ECH), jnp.int32),
        pltpu.VMEM((2, ECH), jnp.int32),
        pltpu.VMEM((2, ECH, D), jnp.float32),
        pltpu.VMEM((ECH, D), jnp.float32),
        pltpu.VMEM_SHARED((NPAD, D), jnp.float32),
        pltpu.SemaphoreType.DMA,
        pltpu.SemaphoreType.DMA,
    ],
)
def _agg(g_hbm, src_hbm, dst_hbm, out_hbm, sidx, didx, rows, ztile, acc, sem0,
         sem1):
    c = lax.axis_index("c")
    s = lax.axis_index("s")
    wid = c * NS + s
    base = wid * EPT
    sems = (sem0, sem1)

    zeros16 = jnp.zeros((16,), jnp.float32)

    def zero_body(i, carry):
        r = i // (D // 16)
        col = (i % (D // 16)) * 16
        ztile[r, pl.ds(col, 16)] = zeros16
        return carry

    lax.fori_loop(0, ECH * (D // 16), zero_body, 0)
    row0 = s * ROWS_PER_TILE
    for j in range(WB):
        pltpu.sync_copy(ztile, acc.at[pl.ds(row0 + j * ECH, ECH)])
    plsc.subcore_barrier()

    def load(k, b):
        pltpu.sync_copy(src_hbm.at[pl.ds(base + k * ECH, ECH)], sidx.at[b])
        pltpu.sync_copy(dst_hbm.at[pl.ds(base + k * ECH, ECH)], didx.at[b])
        pltpu.async_copy(g_hbm.at[sidx.at[b]], rows.at[b], sems[b])

    def wait(b):
        pltpu.make_async_copy(g_hbm.at[sidx.at[b]], rows.at[b], sems[b]).wait()

    def scatter(b):
        pltpu.sync_copy(rows.at[b], acc.at[didx.at[b]], add=True)

    load(0, 0)

    def step(k2, carry):
        load(2 * k2 + 1, 1)
        wait(0)
        scatter(0)

        @pl.when(2 * k2 + 2 < NCH)
        def _():
            load(2 * k2 + 2, 0)

        wait(1)
        scatter(1)
        return carry

    lax.fori_loop(0, NCH // 2, step, 0)

    plsc.subcore_barrier()
    for j in range(WB):
        pltpu.sync_copy(acc.at[pl.ds(row0 + j * ECH, ECH)], rows.at[0])
        pltpu.sync_copy(rows.at[0], out_hbm.at[c, pl.ds(row0 + j * ECH, ECH)])


BR = 1024


def _mm1_body(cnt_ref, x_ref, w_ref, g_ref, dinv_ref):
    deg = jnp.sum(cnt_ref[...], axis=0) + 1.0
    dinv = lax.rsqrt(deg)
    h = jnp.dot(x_ref[...], w_ref[...], preferred_element_type=jnp.float32)
    g_ref[...] = h * dinv[:, None]
    dinv_ref[...] = dinv


def _mm1(counts, x_p, W1):
    return pl.pallas_call(
        _mm1_body,
        grid=(NPAD // BR,),
        in_specs=[
            pl.BlockSpec((NW, BR), lambda i: (0, i)),
            pl.BlockSpec((BR, D), lambda i: (i, 0)),
            pl.BlockSpec((D, D), lambda i: (0, 0)),
        ],
        out_specs=[
            pl.BlockSpec((BR, D), lambda i: (i, 0)),
            pl.BlockSpec((BR,), lambda i: (i,)),
        ],
        out_shape=[
            jax.ShapeDtypeStruct((NPAD, D), jnp.float32),
            jax.ShapeDtypeStruct((NPAD,), jnp.float32),
        ],
    )(counts, x_p, W1)


def _mm2_body(p_ref, g_ref, dinv_ref, b_ref, w_ref, g2_ref):
    dinv = dinv_ref[...]
    o = (p_ref[0] + p_ref[1] + g_ref[...]) * dinv[:, None] + b_ref[...]
    h = jnp.maximum(o, 0.0)
    g2_ref[...] = (
        jnp.dot(h, w_ref[...], preferred_element_type=jnp.float32)
        * dinv[:, None]
    )


def _mm2(partials, g1, dinv, b1, W2):
    return pl.pallas_call(
        _mm2_body,
        grid=(NPAD // BR,),
        in_specs=[
            pl.BlockSpec((NC, BR, D), lambda i: (0, i, 0)),
            pl.BlockSpec((BR, D), lambda i: (i, 0)),
            pl.BlockSpec((BR,), lambda i: (i,)),
            pl.BlockSpec((D,), lambda i: (0,)),
            pl.BlockSpec((D, D), lambda i: (0, 0)),
        ],
        out_specs=pl.BlockSpec((BR, D), lambda i: (i, 0)),
        out_shape=jax.ShapeDtypeStruct((NPAD, D), jnp.float32),
    )(partials, g1, dinv, b1, W2)


def _fin_body(p_ref, g_ref, dinv_ref, b_ref, batch_ref, o_ref):
    i = pl.program_id(0)

    @pl.when(i == 0)
    def _():
        o_ref[...] = jnp.full((G, D), -jnp.inf, jnp.float32)

    o = (p_ref[0] + p_ref[1] + g_ref[...]) * dinv_ref[...][:, None] + b_ref[...]
    bt = batch_ref[...]
    lo = jnp.min(bt)
    hi = jnp.minimum(jnp.max(bt), G - 1) + 1

    def seg(gi, carry):
        m = (bt == gi)[:, None]
        mx = jnp.max(jnp.where(m, o, -jnp.inf), axis=0)
        cur = o_ref[pl.ds(gi, 1), :]
        o_ref[pl.ds(gi, 1), :] = jnp.maximum(cur, mx[None, :])
        return carry

    lax.fori_loop(lo, hi, seg, 0)


def _fin(partials, g2, dinv, b2, batch_p):
    return pl.pallas_call(
        _fin_body,
        grid=(NPAD // BR,),
        in_specs=[
            pl.BlockSpec((NC, BR, D), lambda i: (0, i, 0)),
            pl.BlockSpec((BR, D), lambda i: (i, 0)),
            pl.BlockSpec((BR,), lambda i: (i,)),
            pl.BlockSpec((D,), lambda i: (0,)),
            pl.BlockSpec((BR,), lambda i: (i,)),
        ],
        out_specs=pl.BlockSpec((G, D), lambda i: (0, 0)),
        out_shape=jax.ShapeDtypeStruct((G, D), jnp.float32),
    )(partials, g2, dinv, b2, batch_p)


def kernel(x, edge_index, batch, W1, b1, W2, b2):
    src = edge_index[0]
    dst = edge_index[1]
    pad_idx = jnp.full((EPAD - E,), N, jnp.int32)
    src_p = jnp.concatenate([src, pad_idx])
    dst_p = jnp.concatenate([dst, pad_idx])
    x_p = jnp.pad(x.astype(jnp.float32), ((0, NPAD - N), (0, 0)))
    batch_p = jnp.concatenate(
        [batch, jnp.full((NPAD - N,), G, jnp.int32)])

    counts = _hist(dst_p)
    g1, dinv = _mm1(counts, x_p, W1)
    p1 = _agg(g1, src_p, dst_p)
    g2 = _mm2(p1, g1, dinv, b1, W2)
    p2 = _agg(g2, src_p, dst_p)
    return _fin(p2, g2, dinv, b2, batch_p)


# speedup vs baseline: 11.3072x; 11.3072x over previous
import functools

import jax
import jax.numpy as jnp
from jax import lax
from jax.experimental import pallas as pl
from jax.experimental.pallas import tpu as pltpu
from jax.experimental.pallas import tpu_sc as plsc

N = 10000
E = 320000
D = 128
G = 64

NC = 2
NS = 16
NW = NC * NS

NPAD = 10240
EPAD = 327680
EPT = EPAD // NW
ECH = 128
NCH = EPT // ECH
ROWS_PER_TILE = NPAD // NS
WB = ROWS_PER_TILE // ECH

@functools.cache
def _mesh():
    return plsc.VectorSubcoreMesh(
        core_axis_name="c", subcore_axis_name="s",
        num_cores=NC, num_subcores=NS)


HCH = 512


@functools.cache
def _hist():
    return pl.kernel(
        _hist_body,
        out_type=jax.ShapeDtypeStruct((NW, NPAD), jnp.float32),
        mesh=_mesh(),
        scratch_types=[
            pltpu.VMEM((HCH,), jnp.int32),
            pltpu.VMEM((NPAD,), jnp.float32),
        ],
        compiler_params=pltpu.CompilerParams(needs_layout_passes=False),
    )


def _hist_body(dst_hbm, cnt_hbm, didx_v, cnt_v):
    c = lax.axis_index("c")
    s = lax.axis_index("s")
    wid = c * NS + s
    base = wid * EPT

    zeros16 = jnp.zeros((16,), jnp.float32)
    ones16 = jnp.ones((16,), jnp.float32)

    def zero_body(i, carry):
        cnt_v[pl.ds(i * 16, 16)] = zeros16
        return carry

    lax.fori_loop(0, NPAD // 16, zero_body, 0)

    def chunk_body(k, carry):
        pltpu.sync_copy(dst_hbm.at[pl.ds(base + k * HCH, HCH)], didx_v)

        def inner(j, carry2):
            iv = didx_v[pl.ds(j * 16, 16)]
            plsc.addupdate_scatter(cnt_v, [iv], ones16)
            return carry2

        lax.fori_loop(0, HCH // 16, inner, 0)
        return carry

    lax.fori_loop(0, EPT // HCH, chunk_body, 0)
    pltpu.sync_copy(cnt_v, cnt_hbm.at[wid])


@functools.cache
def _agg():
    return pl.kernel(
        _agg_body,
        out_type=jax.ShapeDtypeStruct((NC, NPAD, D), jnp.float32),
        mesh=_mesh(),
        scratch_types=[
            pltpu.VMEM((2, ECH), jnp.int32),
            pltpu.VMEM((2, ECH), jnp.int32),
            pltpu.VMEM((2, ECH, D), jnp.float32),
            pltpu.VMEM_SHARED((NPAD, D), jnp.float32),
            pltpu.SemaphoreType.DMA,
            pltpu.SemaphoreType.DMA,
        ],
        compiler_params=pltpu.CompilerParams(needs_layout_passes=False),
    )


def _agg_body(g_hbm, src_hbm, dst_hbm, out_hbm, sidx, didx, rows, acc,
              sem0, sem1):
    c = lax.axis_index("c")
    s = lax.axis_index("s")
    wid = c * NS + s
    base = wid * EPT
    sems = (sem0, sem1)

    zeros16 = jnp.zeros((16,), jnp.float32)

    def zero_body(i, carry):
        r = i // (D // 16)
        col = (i % (D // 16)) * 16
        rows[0, r, pl.ds(col, 16)] = zeros16
        return carry

    lax.fori_loop(0, ECH * (D // 16), zero_body, 0)
    row0 = s * ROWS_PER_TILE
    for j in range(WB):
        pltpu.sync_copy(rows.at[0], acc.at[pl.ds(row0 + j * ECH, ECH)])
    plsc.subcore_barrier()

    def load(k, b):
        pltpu.sync_copy(src_hbm.at[pl.ds(base + k * ECH, ECH)], sidx.at[b])
        pltpu.sync_copy(dst_hbm.at[pl.ds(base + k * ECH, ECH)], didx.at[b])
        pltpu.async_copy(g_hbm.at[sidx.at[b]], rows.at[b], sems[b])

    def wait(b):
        pltpu.make_async_copy(g_hbm.at[sidx.at[b]], rows.at[b], sems[b]).wait()

    def scatter(b):
        pltpu.sync_copy(rows.at[b], acc.at[didx.at[b]], add=True)

    load(0, 0)

    def step(k2, carry):
        load(2 * k2 + 1, 1)
        wait(0)
        scatter(0)

        @pl.when(2 * k2 + 2 < NCH)
        def _():
            load(2 * k2 + 2, 0)

        wait(1)
        scatter(1)
        return carry

    lax.fori_loop(0, NCH // 2, step, 0)

    plsc.subcore_barrier()
    for j in range(WB):
        pltpu.sync_copy(acc.at[pl.ds(row0 + j * ECH, ECH)], rows.at[0])
        pltpu.sync_copy(rows.at[0], out_hbm.at[c, pl.ds(row0 + j * ECH, ECH)])


BR = 1024


def _mm1_body(cnt_ref, x_ref, w_ref, g_ref, dinv_ref):
    deg = jnp.sum(cnt_ref[...], axis=0) + 1.0
    dinv = lax.rsqrt(deg)
    h = jnp.dot(x_ref[...], w_ref[...], preferred_element_type=jnp.float32)
    g_ref[...] = h * dinv[:, None]
    dinv_ref[...] = dinv


def _mm1(counts, x_p, W1):
    return pl.pallas_call(
        _mm1_body,
        grid=(NPAD // BR,),
        in_specs=[
            pl.BlockSpec((NW, BR), lambda i: (0, i)),
            pl.BlockSpec((BR, D), lambda i: (i, 0)),
            pl.BlockSpec((D, D), lambda i: (0, 0)),
        ],
        out_specs=[
            pl.BlockSpec((BR, D), lambda i: (i, 0)),
            pl.BlockSpec((BR,), lambda i: (i,)),
        ],
        out_shape=[
            jax.ShapeDtypeStruct((NPAD, D), jnp.float32),
            jax.ShapeDtypeStruct((NPAD,), jnp.float32),
        ],
    )(counts, x_p, W1)


def _mm2_body(p_ref, g_ref, dinv_ref, b_ref, w_ref, g2_ref):
    dinv = dinv_ref[...]
    o = (p_ref[0] + p_ref[1] + g_ref[...]) * dinv[:, None] + b_ref[...]
    h = jnp.maximum(o, 0.0)
    g2_ref[...] = (
        jnp.dot(h, w_ref[...], preferred_element_type=jnp.float32)
        * dinv[:, None]
    )


def _mm2(partials, g1, dinv, b1, W2):
    return pl.pallas_call(
        _mm2_body,
        grid=(NPAD // BR,),
        in_specs=[
            pl.BlockSpec((NC, BR, D), lambda i: (0, i, 0)),
            pl.BlockSpec((BR, D), lambda i: (i, 0)),
            pl.BlockSpec((BR,), lambda i: (i,)),
            pl.BlockSpec((D,), lambda i: (0,)),
            pl.BlockSpec((D, D), lambda i: (0, 0)),
        ],
        out_specs=pl.BlockSpec((BR, D), lambda i: (i, 0)),
        out_shape=jax.ShapeDtypeStruct((NPAD, D), jnp.float32),
    )(partials, g1, dinv, b1, W2)


def _fin_body(p_ref, g_ref, dinv_ref, b_ref, batch_ref, o_ref):
    i = pl.program_id(0)

    @pl.when(i == 0)
    def _():
        o_ref[...] = jnp.full((G, D), -jnp.inf, jnp.float32)

    o = (p_ref[0] + p_ref[1] + g_ref[...]) * dinv_ref[...][:, None] + b_ref[...]
    bt = batch_ref[...]
    lo = jnp.min(bt)
    hi = jnp.minimum(jnp.max(bt), G - 1) + 1

    def seg(gi, carry):
        mf = jnp.where(bt == gi, 0.0, -jnp.inf)
        mx = jnp.max(o + mf[:, None], axis=0)
        cur = o_ref[pl.ds(gi, 1), :]
        o_ref[pl.ds(gi, 1), :] = jnp.maximum(cur, mx[None, :])
        return carry

    lax.fori_loop(lo, hi, seg, 0)


def _fin(partials, g2, dinv, b2, batch_p):
    return pl.pallas_call(
        _fin_body,
        grid=(NPAD // BR,),
        in_specs=[
            pl.BlockSpec((NC, BR, D), lambda i: (0, i, 0)),
            pl.BlockSpec((BR, D), lambda i: (i, 0)),
            pl.BlockSpec((BR,), lambda i: (i,)),
            pl.BlockSpec((D,), lambda i: (0,)),
            pl.BlockSpec((BR,), lambda i: (i,)),
        ],
        out_specs=pl.BlockSpec((G, D), lambda i: (0, 0)),
        out_shape=jax.ShapeDtypeStruct((G, D), jnp.float32),
    )(partials, g2, dinv, b2, batch_p)


def kernel(x, edge_index, batch, W1, b1, W2, b2):
    src = edge_index[0]
    dst = edge_index[1]
    pad_idx = jnp.full((EPAD - E,), N, jnp.int32)
    src_p = jnp.concatenate([src, pad_idx])
    dst_p = jnp.concatenate([dst, pad_idx])
    x_p = jnp.pad(x.astype(jnp.float32), ((0, NPAD - N), (0, 0)))
    batch_p = jnp.concatenate(
        [batch, jnp.full((NPAD - N,), G, jnp.int32)])

    counts = _hist()(dst_p)
    g1, dinv = _mm1(counts, x_p, W1)
    p1 = _agg()(g1, src_p, dst_p)
    g2 = _mm2(p1, g1, dinv, b1, W2)
    p2 = _agg()(g2, src_p, dst_p)
    return _fin(p2, g2, dinv, b2, batch_p)


# speedup vs baseline: 11.6112x; 1.0269x over previous
import functools

import jax
import jax.numpy as jnp
from jax import lax
from jax.experimental import pallas as pl
from jax.experimental.pallas import tpu as pltpu
from jax.experimental.pallas import tpu_sc as plsc

N = 10000
E = 320000
D = 128
G = 64

NC = 2
NS = 16
NW = NC * NS

NPAD = 10240
EPAD = 327680
EPT = EPAD // NW
ECH = 128
NCH = EPT // ECH
ROWS_PER_TILE = NPAD // NS
WB = ROWS_PER_TILE // ECH

@functools.cache
def _mesh():
    return plsc.VectorSubcoreMesh(
        core_axis_name="c", subcore_axis_name="s",
        num_cores=NC, num_subcores=NS)


@functools.cache
def _hist():
    return pl.kernel(
        _hist_body,
        out_type=jax.ShapeDtypeStruct((NW, NPAD), jnp.float32),
        mesh=_mesh(),
        scratch_types=[
            pltpu.VMEM((EPT,), jnp.int32),
            pltpu.VMEM((NPAD,), jnp.float32),
        ],
        compiler_params=pltpu.CompilerParams(needs_layout_passes=False),
    )


def _hist_body(dst_hbm, cnt_hbm, didx_v, cnt_v):
    c = lax.axis_index("c")
    s = lax.axis_index("s")
    wid = c * NS + s
    base = wid * EPT

    zeros16 = jnp.zeros((16,), jnp.float32)
    ones16 = jnp.ones((16,), jnp.float32)

    def zero_body(i, carry):
        cnt_v[pl.ds(i * 16, 16)] = zeros16
        return carry

    lax.fori_loop(0, NPAD // 16, zero_body, 0)
    pltpu.sync_copy(dst_hbm.at[pl.ds(base, EPT)], didx_v)

    def inner(j, carry2):
        iv = didx_v[pl.ds(j * 16, 16)]
        plsc.addupdate_scatter(cnt_v, [iv], ones16)
        return carry2

    lax.fori_loop(0, EPT // 16, inner, 0)
    pltpu.sync_copy(cnt_v, cnt_hbm.at[wid])


NPH = 2
CPP = NCH // NPH


@functools.cache
def _agg():
    return pl.kernel(
        _agg_body,
        out_type=jax.ShapeDtypeStruct((NC, NPAD, D), jnp.float32),
        mesh=_mesh(),
        scratch_types=[
            pltpu.VMEM((CPP, ECH), jnp.int32),
            pltpu.VMEM((CPP, ECH), jnp.int32),
            pltpu.VMEM((2, ECH, D), jnp.float32),
            pltpu.VMEM_SHARED((NPAD, D), jnp.float32),
            pltpu.SemaphoreType.DMA,
            pltpu.SemaphoreType.DMA,
        ],
        compiler_params=pltpu.CompilerParams(needs_layout_passes=False),
    )


def _agg_body(g_hbm, src_hbm, dst_hbm, out_hbm, sidx, didx, rows, acc,
              sem0, sem1):
    c = lax.axis_index("c")
    s = lax.axis_index("s")
    wid = c * NS + s
    sems = (sem0, sem1)

    zeros16 = jnp.zeros((16,), jnp.float32)

    def zero_body(i, carry):
        r = i // (D // 16)
        col = (i % (D // 16)) * 16
        rows[0, r, pl.ds(col, 16)] = zeros16
        return carry

    lax.fori_loop(0, ECH * (D // 16), zero_body, 0)
    row0 = s * ROWS_PER_TILE
    for j in range(WB):
        pltpu.sync_copy(rows.at[0], acc.at[pl.ds(row0 + j * ECH, ECH)])
    plsc.subcore_barrier()

    def fire(k, b):
        pltpu.async_copy(g_hbm.at[sidx.at[k]], rows.at[b], sems[b])

    def wait(b):
        pltpu.make_async_copy(g_hbm.at[sidx.at[0]], rows.at[b], sems[b]).wait()

    def scatter(k, b):
        pltpu.sync_copy(rows.at[b], acc.at[didx.at[k]], add=True)

    for ph in range(NPH):
        pltpu.sync_copy(src_hbm.at[wid, pl.ds(ph * CPP, CPP)], sidx)
        pltpu.sync_copy(dst_hbm.at[wid, pl.ds(ph * CPP, CPP)], didx)
        fire(0, 0)

        def step(k2, carry):
            fire(2 * k2 + 1, 1)
            wait(0)
            scatter(2 * k2, 0)

            @pl.when(2 * k2 + 2 < CPP)
            def _():
                fire(2 * k2 + 2, 0)

            wait(1)
            scatter(2 * k2 + 1, 1)
            return carry

        lax.fori_loop(0, CPP // 2, step, 0)

    plsc.subcore_barrier()
    for j in range(WB):
        pltpu.sync_copy(acc.at[pl.ds(row0 + j * ECH, ECH)], rows.at[0])
        pltpu.sync_copy(rows.at[0], out_hbm.at[c, pl.ds(row0 + j * ECH, ECH)])


BR = 1024


def _mm1_body(cnt_ref, x_ref, w_ref, g_ref, dinv_ref):
    deg = jnp.sum(cnt_ref[...], axis=0) + 1.0
    dinv = lax.rsqrt(deg)
    h = jnp.dot(x_ref[...], w_ref[...], preferred_element_type=jnp.float32)
    g_ref[...] = h * dinv[:, None]
    dinv_ref[...] = dinv


def _mm1(counts, x_p, W1):
    return pl.pallas_call(
        _mm1_body,
        grid=(NPAD // BR,),
        in_specs=[
            pl.BlockSpec((NW, BR), lambda i: (0, i)),
            pl.BlockSpec((BR, D), lambda i: (i, 0)),
            pl.BlockSpec((D, D), lambda i: (0, 0)),
        ],
        out_specs=[
            pl.BlockSpec((BR, D), lambda i: (i, 0)),
            pl.BlockSpec((BR,), lambda i: (i,)),
        ],
        out_shape=[
            jax.ShapeDtypeStruct((NPAD, D), jnp.float32),
            jax.ShapeDtypeStruct((NPAD,), jnp.float32),
        ],
    )(counts, x_p, W1)


def _mm2_body(p_ref, g_ref, dinv_ref, b_ref, w_ref, g2_ref):
    dinv = dinv_ref[...]
    o = (p_ref[0] + p_ref[1] + g_ref[...]) * dinv[:, None] + b_ref[...]
    h = jnp.maximum(o, 0.0)
    g2_ref[...] = (
        jnp.dot(h, w_ref[...], preferred_element_type=jnp.float32)
        * dinv[:, None]
    )


def _mm2(partials, g1, dinv, b1, W2):
    return pl.pallas_call(
        _mm2_body,
        grid=(NPAD // BR,),
        in_specs=[
            pl.BlockSpec((NC, BR, D), lambda i: (0, i, 0)),
            pl.BlockSpec((BR, D), lambda i: (i, 0)),
            pl.BlockSpec((BR,), lambda i: (i,)),
            pl.BlockSpec((D,), lambda i: (0,)),
            pl.BlockSpec((D, D), lambda i: (0, 0)),
        ],
        out_specs=pl.BlockSpec((BR, D), lambda i: (i, 0)),
        out_shape=jax.ShapeDtypeStruct((NPAD, D), jnp.float32),
    )(partials, g1, dinv, b1, W2)


def _fin_body(p_ref, g_ref, dinv_ref, b_ref, batch_ref, o_ref):
    i = pl.program_id(0)

    @pl.when(i == 0)
    def _():
        o_ref[...] = jnp.full((G, D), -jnp.inf, jnp.float32)

    o = (p_ref[0] + p_ref[1] + g_ref[...]) * dinv_ref[...][:, None] + b_ref[...]
    bt = batch_ref[...]
    lo = jnp.min(bt)
    hi = jnp.minimum(jnp.max(bt), G - 1) + 1

    def seg(gi, carry):
        mf = jnp.where(bt == gi, 0.0, -jnp.inf)
        mx = jnp.max(o + mf[:, None], axis=0)
        cur = o_ref[pl.ds(gi, 1), :]
        o_ref[pl.ds(gi, 1), :] = jnp.maximum(cur, mx[None, :])
        return carry

    lax.fori_loop(lo, hi, seg, 0)


def _fin(partials, g2, dinv, b2, batch_p):
    return pl.pallas_call(
        _fin_body,
        grid=(NPAD // BR,),
        in_specs=[
            pl.BlockSpec((NC, BR, D), lambda i: (0, i, 0)),
            pl.BlockSpec((BR, D), lambda i: (i, 0)),
            pl.BlockSpec((BR,), lambda i: (i,)),
            pl.BlockSpec((D,), lambda i: (0,)),
            pl.BlockSpec((BR,), lambda i: (i,)),
        ],
        out_specs=pl.BlockSpec((G, D), lambda i: (0, 0)),
        out_shape=jax.ShapeDtypeStruct((G, D), jnp.float32),
    )(partials, g2, dinv, b2, batch_p)


def kernel(x, edge_index, batch, W1, b1, W2, b2):
    src = edge_index[0]
    dst = edge_index[1]
    pad_idx = jnp.full((EPAD - E,), N, jnp.int32)
    src_f = jnp.concatenate([src, pad_idx])
    dst_f = jnp.concatenate([dst, pad_idx])
    src_p = src_f.reshape(NW, NCH, ECH)
    dst_p = dst_f.reshape(NW, NCH, ECH)
    x_p = jnp.pad(x.astype(jnp.float32), ((0, NPAD - N), (0, 0)))
    batch_p = jnp.concatenate(
        [batch, jnp.full((NPAD - N,), G, jnp.int32)])

    counts = _hist()(dst_f)
    g1, dinv = _mm1(counts, x_p, W1)
    p1 = _agg()(g1, src_p, dst_p)
    g2 = _mm2(p1, g1, dinv, b1, W2)
    p2 = _agg()(g2, src_p, dst_p)
    return _fin(p2, g2, dinv, b2, batch_p)
